# baseline (speedup 1.0000x reference)
import functools

import jax
import jax.numpy as jnp
from jax import lax
from jax.experimental import pallas as pl
from jax.experimental.pallas import tpu as pltpu
from jax.experimental.pallas import tpu_sc as plsc

EMBED_DIM = 512
TIMESTEPS = 1000
TBL = 1024
BATCH = 16384

NC = 2
NS = 16
NW = NC * NS
BPW = BATCH // NW
CH = 128
NCHUNK = BPW // CH


def _mlp_table_body(enc_ref, w1_ref, b1_ref, w2_ref, b2_ref, out_ref):
    h = jnp.dot(enc_ref[...], w1_ref[...], preferred_element_type=jnp.float32)
    h = h + b1_ref[...]
    h = jnp.where(h >= 0, h, 0.01 * h)
    o = jnp.dot(h, w2_ref[...], preferred_element_type=jnp.float32)
    out_ref[pl.ds(0, TIMESTEPS), :] = o + b2_ref[...]


def _compute_table(encoding, W1, b1, W2, b2):
    return pl.pallas_call(
        _mlp_table_body,
        out_shape=jax.ShapeDtypeStruct((TBL, EMBED_DIM), jnp.float32),
    )(encoding, W1, b1.reshape(1, EMBED_DIM), W2, b2.reshape(1, EMBED_DIM))


def _gather_body(table_hbm, idx_hbm, out_hbm, idx_v, rows, gsem, wsem):
    s = lax.axis_index("s")
    wid = s * NC + lax.axis_index("c")
    base = wid * BPW
    pltpu.sync_copy(idx_hbm.at[wid], idx_v)
    wh = None
    for j in range(NCHUNK):
        if wh is not None:
            wh.wait()
        pltpu.async_copy(table_hbm.at[idx_v.at[j]], rows, gsem).wait()
        wh = pltpu.async_copy(rows, out_hbm.at[pl.ds(base + j * CH, CH)], wsem)
    wh.wait()


_gather = functools.partial(
    pl.kernel,
    out_type=jax.ShapeDtypeStruct((BATCH, EMBED_DIM), jnp.float32),
    mesh=plsc.VectorSubcoreMesh(core_axis_name="c", subcore_axis_name="s"),
    scratch_types=[
        pltpu.VMEM((NCHUNK, CH), jnp.int32),
        pltpu.VMEM((CH, EMBED_DIM), jnp.float32),
        pltpu.SemaphoreType.DMA,
        pltpu.SemaphoreType.DMA,
    ],
)(_gather_body)


def kernel(t, encoding, W1, b1, W2, b2):
    table = _compute_table(encoding, W1, b1, W2, b2)
    idx = t.astype(jnp.int32).reshape(NW, NCHUNK, CH)
    return _gather(table, idx)
